# baseline (speedup 1.0000x reference)
import functools

import jax
import jax.numpy as jnp
from jax import lax
from jax.experimental import pallas as pl
from jax.experimental.pallas import tpu as pltpu
from jax.experimental.pallas import tpu_sc as plsc

N = 10000
DIN = 128
DOUT = 16

NC = 2
NS = 16
RT = 20
R_SC = NC * NS * RT
BASE = N - R_SC
TILE = 400
NT_TC = -(-BASE // TILE)
CH = 1280
CHUNKS = [(c, CH) for c in range(0, 7 * CH, CH)] + [(8960, 1024)]
TAIL = 9984


def _prologue_body(x_ref, w_ref, b_ref, h_ref):
    h_ref[...] = jax.nn.relu(
        jnp.dot(x_ref[...], w_ref[...], preferred_element_type=jnp.float32)
        + b_ref[...]
    )


def _prologue(x, W, b2):
    return pl.pallas_call(
        _prologue_body,
        out_shape=jax.ShapeDtypeStruct((N, DOUT), jnp.float32),
    )(x, W, b2)


def _tc_pass_body(g_ref, h_ref, o_ref):
    o_ref[...] = jnp.dot(g_ref[...], h_ref[...], preferred_element_type=jnp.float32)


def _tc_pass(g, h):
    return pl.pallas_call(
        _tc_pass_body,
        grid=(NT_TC,),
        in_specs=[
            pl.BlockSpec((TILE, N), lambda i: (i, 0)),
            pl.BlockSpec((N, DOUT), lambda i: (0, 0)),
        ],
        out_specs=pl.BlockSpec((TILE, DOUT), lambda i: (i, 0)),
        out_shape=jax.ShapeDtypeStruct((BASE, DOUT), jnp.float32),
        compiler_params=pltpu.CompilerParams(
            dimension_semantics=("arbitrary",),
        ),
    )(g, h)


@functools.partial(
    pl.kernel,
    out_type=jax.ShapeDtypeStruct((NC * NS, RT, DOUT), jnp.float32),
    mesh=plsc.VectorSubcoreMesh(
        core_axis_name="c", subcore_axis_name="s", num_cores=NC, num_subcores=NS
    ),
    scratch_types=[
        pltpu.VMEM((2, CH // 8, 128), jnp.float32),
        pltpu.VMEM((2 * RT * CH,), jnp.float32),
        pltpu.VMEM((RT, DOUT), jnp.float32),
        pltpu.SemaphoreType.DMA,
        pltpu.SemaphoreType.DMA,
    ],
)
def _sc_pass(g_hbm, hr_hbm, out_hbm, h_v, g_v, acc_v, sem0, sem1):
    wid = lax.axis_index("c") * NS + lax.axis_index("s")
    row0 = BASE + wid * RT
    sems = (sem0, sem1)

    def start(i):
        c0, clen = CHUNKS[i]
        s = i % 2
        cops = [
            pltpu.async_copy(
                hr_hbm.at[pl.ds(c0 // 8, clen // 8)],
                h_v.at[s, pl.ds(0, clen // 8)],
                sems[s],
            )
        ]
        for r in range(RT):
            cops.append(
                pltpu.async_copy(
                    g_hbm.at[row0 + r, pl.ds(c0, clen)],
                    g_v.at[pl.ds((s * RT + r) * CH, clen)],
                    sems[s],
                )
            )
        return cops

    for r in range(RT):
        acc_v[r, :] = jnp.zeros((DOUT,), jnp.float32)

    pending = start(0)
    for i, (c0, clen) in enumerate(CHUNKS):
        nxt = start(i + 1) if i + 1 < len(CHUNKS) else []
        for c in pending:
            c.wait()
        pending = nxt
        s = i % 2

        def jbody(jb, carry, s=s):
            hvs = [
                h_v[s, jb * 2 + (l // 8), pl.ds((l % 8) * 16, 16)]
                for l in range(16)
            ]
            for rg in range(0, RT, 10):
                gvs = [
                    g_v[pl.ds((s * RT + rg + q) * CH + jb * 16, 16)]
                    for q in range(10)
                ]
                for l in range(0, 16, 2):
                    for q in range(10):
                        plsc.addupdate(
                            acc_v.at[rg + q, :],
                            gvs[q][l] * hvs[l] + gvs[q][l + 1] * hvs[l + 1],
                        )
            return carry

        lax.fori_loop(0, clen // 16, jbody, 0)

    pltpu.sync_copy(acc_v, out_hbm.at[wid])


def _tail_body(g_ref, h_ref, o_ref):
    gm = jnp.where(
        jax.lax.broadcasted_iota(jnp.int32, (16, 128), 1) < (N - TAIL),
        g_ref[...],
        0.0,
    )
    o_ref[...] = jnp.dot(gm, h_ref[...], preferred_element_type=jnp.float32)


def _tail_corr(g, h):
    return pl.pallas_call(
        _tail_body,
        grid=(R_SC // 16,),
        in_specs=[
            pl.BlockSpec((16, 128), lambda i: (BASE // 16 + i, TAIL // 128)),
            pl.BlockSpec((128, DOUT), lambda i: (TAIL // 128, 0)),
        ],
        out_specs=pl.BlockSpec((16, DOUT), lambda i: (i, 0)),
        out_shape=jax.ShapeDtypeStruct((R_SC, DOUT), jnp.float32),
    )(g, h)


def _propagate(g, h):
    top = _tc_pass(g, h)
    hr = jnp.reshape(h, (N // 8, 128))
    bot = _sc_pass(g, hr).reshape(R_SC, DOUT) + _tail_corr(g, h)
    return jnp.concatenate([top, bot], axis=0)


@jax.jit
def kernel(x, g, W, b):
    h = _prologue(x, W, b.reshape(1, DOUT))
    h = _propagate(g, h)
    h = _propagate(g, h)
    return h


# speedup vs baseline: 1.6839x; 1.6839x over previous
import functools

import jax
import jax.numpy as jnp
from jax.experimental import pallas as pl
from jax.experimental.pallas import tpu as pltpu

N = 10000
DIN = 128
DOUT = 16
TILE = 400
NT = N // TILE


def _sgc_kernel(x_ref, w_ref, b_ref, g_ref, o_ref, h0_ref, h1_ref):
    k = pl.program_id(0)
    i = pl.program_id(1)

    @pl.when((k == 0) & (i == 0))
    def _prologue():
        h0_ref[...] = jax.nn.relu(
            jnp.dot(x_ref[...], w_ref[...], preferred_element_type=jnp.float32)
            + b_ref[...]
        )

    @pl.when(k == 0)
    def _pass1():
        t = jnp.dot(g_ref[...], h0_ref[...], preferred_element_type=jnp.float32)
        h1_ref[pl.ds(i * TILE, TILE), :] = t
        o_ref[...] = t

    @pl.when(k == 1)
    def _pass2():
        o_ref[...] = jnp.dot(
            g_ref[...], h1_ref[...], preferred_element_type=jnp.float32
        )


@functools.partial(jax.jit, static_argnames=())
def kernel(x, g, W, b):
    b2 = b.reshape(1, DOUT)
    return pl.pallas_call(
        _sgc_kernel,
        grid=(2, NT),
        in_specs=[
            pl.BlockSpec((N, DIN), lambda k, i: (0, 0)),
            pl.BlockSpec((DIN, DOUT), lambda k, i: (0, 0)),
            pl.BlockSpec((1, DOUT), lambda k, i: (0, 0)),
            pl.BlockSpec((TILE, N), lambda k, i: (i, 0)),
        ],
        out_specs=pl.BlockSpec((TILE, DOUT), lambda k, i: (i, 0)),
        out_shape=jax.ShapeDtypeStruct((N, DOUT), jnp.float32),
        scratch_shapes=[
            pltpu.VMEM((N, DOUT), jnp.float32),
            pltpu.VMEM((N, DOUT), jnp.float32),
        ],
        compiler_params=pltpu.CompilerParams(
            dimension_semantics=("arbitrary", "arbitrary"),
            vmem_limit_bytes=120 * 1024 * 1024,
        ),
    )(x, W, b2, g)
